# Initial kernel scaffold; baseline (speedup 1.0000x reference)
import jax
import jax.numpy as jnp
from jax.experimental import pallas as pl


def kernel(node_rep, edge_rep, node2edge_index, W_lvl1, g_lvl1, b_lvl1, W_lvl2a, g_lvl2a, b_lvl2a, W_lvl2b, g_lvl2b, b_lvl2b, W_lifta, g_lifta, b_lifta, W_liftb, g_liftb, b_liftb, eps1, eps2):
    raise NotImplementedError("write your pallas kernel here")


# speedup vs baseline: 1.7227x; 1.7227x over previous
import functools

import jax
import jax.numpy as jnp
from jax import lax
from jax.experimental import pallas as pl
from jax.experimental.pallas import tpu as pltpu

N = 10000
E = 320000
M = 640000
H = 128
EPS_BN = 1e-5

BE = 3200
NB = E // BE


def _dot_t(x, w):
    return lax.dot_general(x, w, (((1,), (1,)), ((), ())),
                           preferred_element_type=jnp.float32)


def _dot(x, w):
    return lax.dot_general(x, w, (((1,), (0,)), ((), ())),
                           preferred_element_type=jnp.float32)


def _p_kernel(node_ref, wa_ref, cnt0_ref, p_ref, sp2_ref):
    p = _dot_t(node_ref[...], wa_ref[...])
    p_ref[...] = p
    sp2_ref[...] = _dot(cnt0_ref[...], p * p)


def _run_p(node_rep, Wa, cnt0p):
    return pl.pallas_call(
        _p_kernel,
        out_shape=(jax.ShapeDtypeStruct((N, H), jnp.float32),
                   jax.ShapeDtypeStruct((8, H), jnp.float32)),
    )(node_rep, Wa, cnt0p)


def _q_kernel(edge_ref, wb_ref, q_ref):
    q_ref[...] = _dot_t(edge_ref[...], wb_ref[...])


def _run_q(edge_rep, Wb):
    return pl.pallas_call(
        _q_kernel,
        grid=(NB,),
        in_specs=[pl.BlockSpec((BE, H), lambda i: (i, 0)),
                  pl.BlockSpec((H, H), lambda i: (0, 0))],
        out_specs=pl.BlockSpec((BE, H), lambda i: (i, 0)),
        out_shape=jax.ShapeDtypeStruct((E, H), jnp.float32),
    )(edge_rep, Wb)


def _e1_kernel(edge_ref, lift_ref, q_ref, cnt1_ref, wla_ref, eps2_ref,
               z1_ref, stats1_ref, sums_ref, g_ref,
               acc1_ref, accs_ref, accg_ref):
    i = pl.program_id(0)

    @pl.when(i == 0)
    def _():
        acc1_ref[...] = jnp.zeros_like(acc1_ref)
        accs_ref[...] = jnp.zeros_like(accs_ref)
        accg_ref[...] = jnp.zeros_like(accg_ref)

    e = edge_ref[...]
    l = lift_ref[...]
    q = q_ref[...]
    w = cnt1_ref[0]
    y = (1.0 + eps2_ref[0, 0]) * e + l
    z1 = _dot_t(y, wla_ref[...])
    z1_ref[...] = z1

    acc1_ref[0:1, :] += jnp.sum(z1, axis=0, keepdims=True)
    acc1_ref[1:2, :] += jnp.sum(z1 * z1, axis=0, keepdims=True)
    accs_ref[0:1, :] += jnp.sum(l, axis=0, keepdims=True)
    sq = _dot(w, q)
    sq2 = _dot(w, q * q)
    accs_ref[1:2, :] += sq[0:1, :]
    accs_ref[2:3, :] += sq2[0:1, :]
    accg_ref[...] += lax.dot_general(l, q, (((0,), (0,)), ((), ())),
                                     preferred_element_type=jnp.float32)

    @pl.when(i == NB - 1)
    def _():
        stats1_ref[...] = acc1_ref[...]
        sums_ref[...] = accs_ref[...]
        g_ref[...] = accg_ref[...]


def _run_e1(edge_rep, lift, Q, cnt1p, W_lifta, eps2):
    return pl.pallas_call(
        _e1_kernel,
        grid=(NB,),
        in_specs=[pl.BlockSpec((BE, H), lambda i: (i, 0)),
                  pl.BlockSpec((BE, H), lambda i: (i, 0)),
                  pl.BlockSpec((BE, H), lambda i: (i, 0)),
                  pl.BlockSpec((1, 8, BE), lambda i: (i, 0, 0)),
                  pl.BlockSpec((H, H), lambda i: (0, 0)),
                  pl.BlockSpec((1, 1), lambda i: (0, 0))],
        out_specs=(pl.BlockSpec((BE, H), lambda i: (i, 0)),
                   pl.BlockSpec((8, H), lambda i: (0, 0)),
                   pl.BlockSpec((8, H), lambda i: (0, 0)),
                   pl.BlockSpec((H, H), lambda i: (0, 0))),
        out_shape=(jax.ShapeDtypeStruct((E, H), jnp.float32),
                   jax.ShapeDtypeStruct((8, H), jnp.float32),
                   jax.ShapeDtypeStruct((8, H), jnp.float32),
                   jax.ShapeDtypeStruct((H, H), jnp.float32)),
        scratch_shapes=[pltpu.VMEM((8, H), jnp.float32),
                        pltpu.VMEM((8, H), jnp.float32),
                        pltpu.VMEM((H, H), jnp.float32)],
    )(edge_rep, lift, Q, cnt1p, W_lifta, eps2)


def _e2_kernel(z1_ref, ab_ref, w_ref, z2_ref, stats2_ref, acc_ref):
    i = pl.program_id(0)

    @pl.when(i == 0)
    def _():
        acc_ref[...] = jnp.zeros_like(acc_ref)

    a = jnp.maximum(z1_ref[...] * ab_ref[0:1, :] + ab_ref[1:2, :], 0.0)
    z2 = _dot_t(a, w_ref[...])
    z2_ref[...] = z2
    acc_ref[0:1, :] += jnp.sum(z2, axis=0, keepdims=True)
    acc_ref[1:2, :] += jnp.sum(z2 * z2, axis=0, keepdims=True)

    @pl.when(i == NB - 1)
    def _():
        stats2_ref[...] = acc_ref[...]


def _run_e2(z1, ab, W):
    return pl.pallas_call(
        _e2_kernel,
        grid=(NB,),
        in_specs=[pl.BlockSpec((BE, H), lambda i: (i, 0)),
                  pl.BlockSpec((8, H), lambda i: (0, 0)),
                  pl.BlockSpec((H, H), lambda i: (0, 0))],
        out_specs=(pl.BlockSpec((BE, H), lambda i: (i, 0)),
                   pl.BlockSpec((8, H), lambda i: (0, 0))),
        out_shape=(jax.ShapeDtypeStruct((E, H), jnp.float32),
                   jax.ShapeDtypeStruct((8, H), jnp.float32)),
        scratch_shapes=[pltpu.VMEM((8, H), jnp.float32)],
    )(z1, ab, W)


def _e3_kernel(z2_ref, ab_ref, out_ref):
    out_ref[...] = jnp.maximum(z2_ref[...] * ab_ref[0:1, :] + ab_ref[1:2, :],
                               0.0)


def _run_e3(z2, ab):
    return pl.pallas_call(
        _e3_kernel,
        grid=(NB,),
        in_specs=[pl.BlockSpec((BE, H), lambda i: (i, 0)),
                  pl.BlockSpec((8, H), lambda i: (0, 0))],
        out_specs=pl.BlockSpec((BE, H), lambda i: (i, 0)),
        out_shape=jax.ShapeDtypeStruct((E, H), jnp.float32),
    )(z2, ab)


def _node_kernel(node_ref, lvl_ref, w2a_ref, gb2a_ref, w2b_ref, gb2b_ref,
                 eps1_ref, out_ref):
    x = (1.0 + eps1_ref[0, 0]) * node_ref[...] + lvl_ref[...]
    z1 = _dot_t(x, w2a_ref[...])
    m1 = jnp.mean(z1, axis=0, keepdims=True)
    v1 = jnp.mean(z1 * z1, axis=0, keepdims=True) - m1 * m1
    a1 = gb2a_ref[0:1, :] * lax.rsqrt(v1 + EPS_BN)
    a = jnp.maximum(a1 * (z1 - m1) + gb2a_ref[1:2, :], 0.0)
    z2 = _dot_t(a, w2b_ref[...])
    m2 = jnp.mean(z2, axis=0, keepdims=True)
    v2 = jnp.mean(z2 * z2, axis=0, keepdims=True) - m2 * m2
    a2 = gb2b_ref[0:1, :] * lax.rsqrt(v2 + EPS_BN)
    out_ref[...] = jnp.maximum(a2 * (z2 - m2) + gb2b_ref[1:2, :], 0.0)


def _run_node(node_rep, lvl, W2a, gb2a, W2b, gb2b, eps1):
    return pl.pallas_call(
        _node_kernel,
        out_shape=jax.ShapeDtypeStruct((N, H), jnp.float32),
    )(node_rep, lvl, W2a, gb2a, W2b, gb2b, eps1)


def kernel(node_rep, edge_rep, node2edge_index,
           W_lvl1, g_lvl1, b_lvl1,
           W_lvl2a, g_lvl2a, b_lvl2a,
           W_lvl2b, g_lvl2b, b_lvl2b,
           W_lifta, g_lifta, b_lifta,
           W_liftb, g_liftb, b_liftb,
           eps1, eps2):
    i0 = node2edge_index[0]
    i1 = node2edge_index[1]
    Wa = W_lvl1[:, :H]
    Wb = W_lvl1[:, H:]
    eps1_2d = jnp.reshape(eps1.astype(jnp.float32), (1, 1))
    eps2_2d = jnp.reshape(eps2.astype(jnp.float32), (1, 1))

    cnt0 = jnp.zeros((N,), jnp.float32).at[i0].add(1.0)
    cnt1 = jnp.zeros((E,), jnp.float32).at[i1].add(1.0)
    lift = jnp.zeros((E, H), jnp.float32).at[i1].add(
        jnp.take(node_rep, i0, axis=0))

    cnt0p = jnp.zeros((8, N), jnp.float32).at[0].set(cnt0)
    cnt1p = jnp.zeros((NB, 8, BE), jnp.float32).at[:, 0, :].set(
        cnt1.reshape(NB, BE))

    P, sP2 = _run_p(node_rep, Wa, cnt0p)
    Q = _run_q(edge_rep, Wb)
    z1e, stats1, sums, G = _run_e1(edge_rep, lift, Q, cnt1p, W_lifta, eps2_2d)

    sL = sums[0]
    sQ = sums[1]
    sQ2 = sums[2]
    sP = sL @ Wa.T
    mu1 = (sP + sQ) / M
    cross = jnp.sum(Wa * G.T, axis=1)
    ex2 = (sP2[0] + sQ2 + 2.0 * cross) / M
    var1 = ex2 - mu1 * mu1
    alpha1 = g_lvl1 * lax.rsqrt(var1 + EPS_BN)
    beta1 = b_lvl1 - alpha1 * mu1

    msg = jax.nn.relu(alpha1 * (jnp.take(P, i0, axis=0)
                                + jnp.take(Q, i1, axis=0)) + beta1)
    T1 = jnp.zeros((E, H), jnp.float32).at[i1].add(msg)
    lvl = jnp.zeros((N, H), jnp.float32).at[i0].add(
        jnp.take(T1, i1, axis=0) - msg)

    m1 = stats1[0] / E
    v1 = stats1[1] / E - m1 * m1
    a1 = g_lifta * lax.rsqrt(v1 + EPS_BN)
    ab1 = jnp.stack([a1, b_lifta - a1 * m1])
    ab1 = jnp.concatenate([ab1, jnp.zeros((6, H), jnp.float32)], axis=0)
    z2e, stats2 = _run_e2(z1e, ab1, W_liftb)
    m2 = stats2[0] / E
    v2 = stats2[1] / E - m2 * m2
    a2 = g_liftb * lax.rsqrt(v2 + EPS_BN)
    ab2 = jnp.stack([a2, b_liftb - a2 * m2])
    ab2 = jnp.concatenate([ab2, jnp.zeros((6, H), jnp.float32)], axis=0)
    edge_out = _run_e3(z2e, ab2)

    gb2a = jnp.concatenate([jnp.stack([g_lvl2a, b_lvl2a]),
                            jnp.zeros((6, H), jnp.float32)], axis=0)
    gb2b = jnp.concatenate([jnp.stack([g_lvl2b, b_lvl2b]),
                            jnp.zeros((6, H), jnp.float32)], axis=0)
    node_out = _run_node(node_rep, lvl, W_lvl2a, gb2a, W_lvl2b, gb2b, eps1_2d)

    return (node_out, edge_out)
